# baseline (speedup 1.0000x reference)
import functools

import jax
import jax.numpy as jnp
from jax import lax
from jax.experimental import pallas as pl
from jax.experimental.pallas import tpu as pltpu
from jax.experimental.pallas import tpu_sc as plsc

_VOCAB = 100000
_D = 256
_BATCH = 32
_ENC_LEN = 2048
_STEPS = 8
_BOS = 1

_TILE_V = 4096
_NT = (_VOCAB + _TILE_V - 1) // _TILE_V
_CHUNK_V = 512
_LANES = 128
_NSTRIPE = 4
_NQ = 4

_CHUNK = 128


def _ctx_sum_sc(tokens_flat, emb):
    info = plsc.get_sparse_core_info()
    nw = info.num_cores * info.num_subcores
    per_w = _BATCH // nw if nw <= _BATCH else 1
    n_chunks = _ENC_LEN // _CHUNK
    mesh = plsc.VectorSubcoreMesh(core_axis_name="c", subcore_axis_name="s")

    nbuf = 3

    @functools.partial(
        pl.kernel,
        out_type=jax.ShapeDtypeStruct((_BATCH, _D), jnp.float32),
        mesh=mesh,
        scratch_types=(
            [pltpu.VMEM((_ENC_LEN,), jnp.int32)]
            + [pltpu.VMEM((_CHUNK, _D), jnp.float32) for _ in range(nbuf)]
            + [pltpu.VMEM((1, _D), jnp.float32)]
            + [pltpu.SemaphoreType.DMA for _ in range(nbuf)]
        ),
    )
    def k(tok_hbm, emb_hbm, out_hbm, idx_v, r0, r1, r2, acc_v, s0, s1, s2):
        wid = lax.axis_index("s") * info.num_cores + lax.axis_index("c")
        rows = (r0, r1, r2)
        sems = (s0, s1, s2)

        def gather(ci, buf):
            return pltpu.async_copy(
                emb_hbm.at[idx_v.at[pl.ds(ci * _CHUNK, _CHUNK)]],
                rows[buf], sems[buf])

        def accum(buf):
            @pl.loop(0, _CHUNK)
            def _(rr):
                for kk in range(0, _D, 16):
                    plsc.addupdate(acc_v.at[0, pl.ds(kk, 16)],
                                   rows[buf][rr, pl.ds(kk, 16)])

        @pl.loop(0, per_w)
        def _(r):
            b = wid * per_w + r
            pltpu.sync_copy(tok_hbm.at[pl.ds(b * _ENC_LEN, _ENC_LEN)], idx_v)

            @pl.loop(0, _D, step=16)
            def _(kk):
                acc_v[0, pl.ds(kk, 16)] = jnp.zeros((16,), jnp.float32)

            for buf in range(nbuf):
                gather(jnp.int32(buf), buf)

            @pl.loop(0, n_chunks - nbuf, step=nbuf)
            def _(ci):
                for buf in range(nbuf):
                    pltpu.make_async_copy(
                        emb_hbm.at[idx_v.at[pl.ds(0, _CHUNK)]],
                        rows[buf], sems[buf]).wait()
                    accum(buf)

                    @pl.when(ci + buf + nbuf < n_chunks)
                    def _():
                        gather(ci + buf + nbuf, buf)

            for buf in range(n_chunks % nbuf):
                pltpu.make_async_copy(
                    emb_hbm.at[idx_v.at[pl.ds(0, _CHUNK)]],
                    rows[buf], sems[buf]).wait()
                accum(buf)

            pltpu.sync_copy(acc_v, out_hbm.at[pl.ds(b, 1)])

    return k(tokens_flat, emb)


def _retile_w(w_out):
    def body(w_ref, o_ref):
        o_ref[...] = w_ref[...][None]

    return pl.pallas_call(
        body,
        grid=(_NT,),
        in_specs=[pl.BlockSpec((_D, _TILE_V), lambda j: (0, j))],
        out_specs=pl.BlockSpec((1, _D, _TILE_V), lambda j: (j, 0, 0)),
        out_shape=jax.ShapeDtypeStruct((_NT, _D, _TILE_V), jnp.float32),
    )(w_out)


def _decode_tc(ctx_sum, denom, emb, w_out):
    def body(ctx_ref, den_ref, emb_ref, wa_ref, wb_ref, wc_ref, wd_ref,
             tok_out, h_out,
             h_s, s_s, rows, best, besti, tok_vmem, tok_smem, sem):
        w_refs = (wa_ref, wb_ref, wc_ref, wd_ref)
        t = pl.program_id(0)
        j = pl.program_id(1)

        @pl.when(jnp.logical_and(t == 0, j == 0))
        def _():
            for b in range(_BATCH):
                tok_smem[b, 0] = _BOS

        @pl.when(j == 0)
        def _():
            copies = []
            for b in range(_BATCH):
                c = pltpu.make_async_copy(emb_ref.at[tok_smem[b, 0]], rows.at[b], sem)
                c.start()
                copies.append(c)
            for c in copies:
                c.wait()

            @pl.when(t == 0)
            def _():
                s_s[...] = rows[...]

            @pl.when(t != 0)
            def _():
                s_s[...] = s_s[...] + rows[...]

            inv = 1.0 / (t.astype(jnp.float32) + 1.0)
            h_s[...] = s_s[...] * inv + ctx_ref[...] / den_ref[...]
            best[...] = jnp.full((_BATCH, _NSTRIPE * _LANES), -jnp.inf, jnp.float32)
            besti[...] = jnp.zeros((_BATCH, _NSTRIPE * _LANES), jnp.int32)

        bvr = [best[:, g * _LANES:(g + 1) * _LANES] for g in range(_NSTRIPE)]
        bcr = [besti[:, g * _LANES:(g + 1) * _LANES] for g in range(_NSTRIPE)]
        lane = lax.broadcasted_iota(jnp.int32, (_BATCH, _LANES), 1)
        for cv in range(_TILE_V // _CHUNK_V):
            lo = cv * _CHUNK_V
            qd = _D // _NQ
            lg = sum(
                jnp.dot(h_s[:, q * qd:(q + 1) * qd],
                        w_refs[q][:, pl.ds(lo, _CHUNK_V)],
                        preferred_element_type=jnp.float32)
                for q in range(_NQ))
            for c in range(_CHUNK_V // _LANES):
                g = (cv * (_CHUNK_V // _LANES) + c) % _NSTRIPE
                v = lg[:, c * _LANES:(c + 1) * _LANES]
                colv = lane + (j * _TILE_V + lo + c * _LANES)
                upd = jnp.logical_and(v > bvr[g], colv < _VOCAB)
                bvr[g] = jnp.where(upd, v, bvr[g])
                bcr[g] = jnp.where(upd, colv, bcr[g])
        best[...] = jnp.concatenate(bvr, axis=1)
        besti[...] = jnp.concatenate(bcr, axis=1)

        @pl.when(j == _NT - 1)
        def _():
            bva = best[...]
            bca = besti[...]
            m = jnp.max(bva, axis=1, keepdims=True)
            idx = jnp.min(jnp.where(bva == m, bca, jnp.int32(2**30)),
                          axis=1, keepdims=True)
            tok_vmem[...] = idx
            tok_out[...] = idx.reshape(1, _BATCH, 1)
            cp = pltpu.make_async_copy(tok_vmem, tok_smem, sem)
            cp.start()
            cp.wait()

            @pl.when(t == _STEPS - 1)
            def _():
                h_out[...] = h_s[...]

    tok_out, h = pl.pallas_call(
        body,
        grid=(_STEPS, _NT),
        in_specs=[
            pl.BlockSpec((_BATCH, _D), lambda t, j: (0, 0)),
            pl.BlockSpec((_BATCH, 1), lambda t, j: (0, 0)),
            pl.BlockSpec(memory_space=pl.ANY),
        ] + [
            pl.BlockSpec((_D // _NQ, _TILE_V),
                         (lambda q: lambda t, j: (q, j))(q))
            for q in range(_NQ)
        ],
        out_specs=[
            pl.BlockSpec((1, _BATCH, 1), lambda t, j: (t, 0, 0)),
            pl.BlockSpec((_BATCH, _D), lambda t, j: (0, 0)),
        ],
        out_shape=[
            jax.ShapeDtypeStruct((_STEPS, _BATCH, 1), jnp.int32),
            jax.ShapeDtypeStruct((_BATCH, _D), jnp.float32),
        ],
        scratch_shapes=[
            pltpu.VMEM((_BATCH, _D), jnp.float32),
            pltpu.VMEM((_BATCH, _D), jnp.float32),
            pltpu.VMEM((_BATCH, _D), jnp.float32),
            pltpu.VMEM((_BATCH, _NSTRIPE * _LANES), jnp.float32),
            pltpu.VMEM((_BATCH, _NSTRIPE * _LANES), jnp.int32),
            pltpu.VMEM((_BATCH, 1), jnp.int32),
            pltpu.SMEM((_BATCH, 1), jnp.int32),
            pltpu.SemaphoreType.DMA,
        ],
    )(ctx_sum, denom, emb, *([w_out] * _NQ))
    return tok_out.reshape(_STEPS, _BATCH), h


def kernel(tokens_enc, enc_masks, emb, W_out):
    ctx_sum = _ctx_sum_sc(tokens_enc.reshape(-1), emb)
    denom = jnp.sum(enc_masks, axis=1, keepdims=True) + 1e-6
    toks, h = _decode_tc(ctx_sum, denom, emb, W_out)
    bos = jnp.full((_BATCH, 1), _BOS, jnp.int32)
    pred_seqs = jnp.concatenate([bos, toks.T], axis=1).astype(tokens_enc.dtype)
    return pred_seqs, h, enc_masks


# speedup vs baseline: 1.0251x; 1.0251x over previous
import functools

import jax
import jax.numpy as jnp
from jax import lax
from jax.experimental import pallas as pl
from jax.experimental.pallas import tpu as pltpu
from jax.experimental.pallas import tpu_sc as plsc

_VOCAB = 100000
_D = 256
_BATCH = 32
_ENC_LEN = 2048
_STEPS = 8
_BOS = 1

_TILE_V = 4096
_NT = (_VOCAB + _TILE_V - 1) // _TILE_V
_CHUNK_V = 512
_LANES = 128
_NSTRIPE = 4
_NQ = 2

_CHUNK = 128


def _ctx_sum_sc(tokens_flat, emb):
    info = plsc.get_sparse_core_info()
    nw = info.num_cores * info.num_subcores
    per_w = _BATCH // nw if nw <= _BATCH else 1
    n_chunks = _ENC_LEN // _CHUNK
    mesh = plsc.VectorSubcoreMesh(core_axis_name="c", subcore_axis_name="s")

    nbuf = 3

    @functools.partial(
        pl.kernel,
        out_type=jax.ShapeDtypeStruct((_BATCH, _D), jnp.float32),
        mesh=mesh,
        scratch_types=(
            [pltpu.VMEM((_ENC_LEN,), jnp.int32)]
            + [pltpu.VMEM((_CHUNK, _D), jnp.float32) for _ in range(nbuf)]
            + [pltpu.VMEM((1, _D), jnp.float32)]
            + [pltpu.SemaphoreType.DMA for _ in range(nbuf)]
        ),
    )
    def k(tok_hbm, emb_hbm, out_hbm, idx_v, r0, r1, r2, acc_v, s0, s1, s2):
        wid = lax.axis_index("s") * info.num_cores + lax.axis_index("c")
        rows = (r0, r1, r2)
        sems = (s0, s1, s2)

        def gather(ci, buf):
            return pltpu.async_copy(
                emb_hbm.at[idx_v.at[pl.ds(ci * _CHUNK, _CHUNK)]],
                rows[buf], sems[buf])

        def accum(buf):
            @pl.loop(0, _CHUNK)
            def _(rr):
                for kk in range(0, _D, 16):
                    plsc.addupdate(acc_v.at[0, pl.ds(kk, 16)],
                                   rows[buf][rr, pl.ds(kk, 16)])

        @pl.loop(0, per_w)
        def _(r):
            b = wid * per_w + r
            pltpu.sync_copy(tok_hbm.at[pl.ds(b * _ENC_LEN, _ENC_LEN)], idx_v)

            @pl.loop(0, _D, step=16)
            def _(kk):
                acc_v[0, pl.ds(kk, 16)] = jnp.zeros((16,), jnp.float32)

            for buf in range(nbuf):
                gather(jnp.int32(buf), buf)

            @pl.loop(0, n_chunks - nbuf, step=nbuf)
            def _(ci):
                for buf in range(nbuf):
                    pltpu.make_async_copy(
                        emb_hbm.at[idx_v.at[pl.ds(0, _CHUNK)]],
                        rows[buf], sems[buf]).wait()
                    accum(buf)

                    @pl.when(ci + buf + nbuf < n_chunks)
                    def _():
                        gather(ci + buf + nbuf, buf)

            for buf in range(n_chunks % nbuf):
                pltpu.make_async_copy(
                    emb_hbm.at[idx_v.at[pl.ds(0, _CHUNK)]],
                    rows[buf], sems[buf]).wait()
                accum(buf)

            pltpu.sync_copy(acc_v, out_hbm.at[pl.ds(b, 1)])

    return k(tokens_flat, emb)


def _retile_w(w_out):
    def body(w_ref, o_ref):
        o_ref[...] = w_ref[...][None]

    return pl.pallas_call(
        body,
        grid=(_NT,),
        in_specs=[pl.BlockSpec((_D, _TILE_V), lambda j: (0, j))],
        out_specs=pl.BlockSpec((1, _D, _TILE_V), lambda j: (j, 0, 0)),
        out_shape=jax.ShapeDtypeStruct((_NT, _D, _TILE_V), jnp.float32),
    )(w_out)


def _decode_tc(ctx_sum, denom, emb, w_out):
    def body(ctx_ref, den_ref, emb_ref, wa_ref, wb_ref,
             tok_out, h_out,
             h_s, s_s, rows, best, besti, tok_vmem, tok_smem, sem):
        w_refs = (wa_ref, wb_ref)
        t = pl.program_id(0)
        j = pl.program_id(1)

        @pl.when(jnp.logical_and(t == 0, j == 0))
        def _():
            for b in range(_BATCH):
                tok_smem[b, 0] = _BOS

        @pl.when(j == 0)
        def _():
            copies = []
            for b in range(_BATCH):
                c = pltpu.make_async_copy(emb_ref.at[tok_smem[b, 0]], rows.at[b], sem)
                c.start()
                copies.append(c)
            for c in copies:
                c.wait()

            @pl.when(t == 0)
            def _():
                s_s[...] = rows[...]

            @pl.when(t != 0)
            def _():
                s_s[...] = s_s[...] + rows[...]

            inv = 1.0 / (t.astype(jnp.float32) + 1.0)
            h_s[...] = s_s[...] * inv + ctx_ref[...] / den_ref[...]
            best[...] = jnp.full((_BATCH, _NSTRIPE * _LANES), -jnp.inf, jnp.float32)
            besti[...] = jnp.zeros((_BATCH, _NSTRIPE * _LANES), jnp.int32)

        bvr = [best[:, g * _LANES:(g + 1) * _LANES] for g in range(_NSTRIPE)]
        bcr = [besti[:, g * _LANES:(g + 1) * _LANES] for g in range(_NSTRIPE)]
        lane = lax.broadcasted_iota(jnp.int32, (_BATCH, _LANES), 1)
        for cv in range(_TILE_V // _CHUNK_V):
            lo = cv * _CHUNK_V
            qd = _D // _NQ
            lg = sum(
                jnp.dot(h_s[:, q * qd:(q + 1) * qd],
                        w_refs[q][:, pl.ds(lo, _CHUNK_V)],
                        preferred_element_type=jnp.float32)
                for q in range(_NQ))
            for c in range(_CHUNK_V // _LANES):
                g = (cv * (_CHUNK_V // _LANES) + c) % _NSTRIPE
                v = lg[:, c * _LANES:(c + 1) * _LANES]
                colv = lane + (j * _TILE_V + lo + c * _LANES)
                upd = jnp.logical_and(v > bvr[g], colv < _VOCAB)
                bvr[g] = jnp.where(upd, v, bvr[g])
                bcr[g] = jnp.where(upd, colv, bcr[g])
        best[...] = jnp.concatenate(bvr, axis=1)
        besti[...] = jnp.concatenate(bcr, axis=1)

        @pl.when(j == _NT - 1)
        def _():
            bva = best[...]
            bca = besti[...]
            m = jnp.max(bva, axis=1, keepdims=True)
            idx = jnp.min(jnp.where(bva == m, bca, jnp.int32(2**30)),
                          axis=1, keepdims=True)
            tok_vmem[...] = idx
            tok_out[...] = idx.reshape(1, _BATCH, 1)
            cp = pltpu.make_async_copy(tok_vmem, tok_smem, sem)
            cp.start()
            cp.wait()

            @pl.when(t == _STEPS - 1)
            def _():
                h_out[...] = h_s[...]

    tok_out, h = pl.pallas_call(
        body,
        grid=(_STEPS, _NT),
        in_specs=[
            pl.BlockSpec((_BATCH, _D), lambda t, j: (0, 0)),
            pl.BlockSpec((_BATCH, 1), lambda t, j: (0, 0)),
            pl.BlockSpec(memory_space=pl.ANY),
        ] + [
            pl.BlockSpec((_D // _NQ, _TILE_V),
                         (lambda q: lambda t, j: (q, j))(q))
            for q in range(_NQ)
        ],
        out_specs=[
            pl.BlockSpec((1, _BATCH, 1), lambda t, j: (t, 0, 0)),
            pl.BlockSpec((_BATCH, _D), lambda t, j: (0, 0)),
        ],
        out_shape=[
            jax.ShapeDtypeStruct((_STEPS, _BATCH, 1), jnp.int32),
            jax.ShapeDtypeStruct((_BATCH, _D), jnp.float32),
        ],
        scratch_shapes=[
            pltpu.VMEM((_BATCH, _D), jnp.float32),
            pltpu.VMEM((_BATCH, _D), jnp.float32),
            pltpu.VMEM((_BATCH, _D), jnp.float32),
            pltpu.VMEM((_BATCH, _NSTRIPE * _LANES), jnp.float32),
            pltpu.VMEM((_BATCH, _NSTRIPE * _LANES), jnp.int32),
            pltpu.VMEM((_BATCH, 1), jnp.int32),
            pltpu.SMEM((_BATCH, 1), jnp.int32),
            pltpu.SemaphoreType.DMA,
        ],
    )(ctx_sum, denom, emb, *([w_out] * _NQ))
    return tok_out.reshape(_STEPS, _BATCH), h


def kernel(tokens_enc, enc_masks, emb, W_out):
    ctx_sum = _ctx_sum_sc(tokens_enc.reshape(-1), emb)
    denom = jnp.sum(enc_masks, axis=1, keepdims=True) + 1e-6
    toks, h = _decode_tc(ctx_sum, denom, emb, W_out)
    bos = jnp.full((_BATCH, 1), _BOS, jnp.int32)
    pred_seqs = jnp.concatenate([bos, toks.T], axis=1).astype(tokens_enc.dtype)
    return pred_seqs, h, enc_masks
